# baseline (speedup 1.0000x reference)
import jax
import jax.numpy as jnp
from jax.experimental import pallas as pl
from jax.experimental.pallas import tpu as pltpu

_TC = 64
_NC = 2
_KT = 256


def _push_tiles(wT):
    for n in (0, 1):
        for k in (0, 1):
            pltpu.matmul_push_rhs(wT[k * _KT:(k + 1) * _KT, n * _KT:(n + 1) * _KT],
                                  staging_register=k, mxu_index=n)


def _gemm_into(scr, x_ref, wT, bias, tc, bh, d, act):
    rows_per_m = _KT // bh
    num_m = (tc * bh) // _KT
    for m in range(num_m):
        xm = x_ref[m * rows_per_m:(m + 1) * rows_per_m].reshape(_KT, d)
        addr = 0 if m % 2 == 0 else 128
        if m > 0:
            _push_tiles(wT)
        for n in (0, 1):
            pltpu.matmul_acc_lhs(addr, xm[:, 0:_KT], mxu_index=n, load_staged_rhs=0)
            pltpu.matmul_acc_lhs(addr, xm[:, _KT:2 * _KT], mxu_index=n, load_staged_rhs=1)
        for n in (0, 1):
            z = pltpu.matmul_pop(addr, (_KT, _KT), jnp.float32, mxu_index=n)
            z = act(z + bias[:, n * _KT:(n + 1) * _KT])
            scr[m * rows_per_m:(m + 1) * rows_per_m, :,
                n * _KT:(n + 1) * _KT] = z.reshape(rows_per_m, bh, _KT)


def _rnn_kernel(x_ref, h0_ref, whT_ref, wxT_ref, bh_ref, wgT_ref, bg_ref,
                out_ref, hlast_ref, h_s, xh_s, g_s):
    tc, bh, d = x_ref.shape
    t_idx = pl.program_id(1)
    n_t = pl.num_programs(1)

    @pl.when(t_idx == 0)
    def _init():
        h_s[...] = h0_ref[...]

    wxT = wxT_ref[...]
    wgT = wgT_ref[...]
    whT = whT_ref[...]

    _push_tiles(wxT)
    _gemm_into(xh_s, x_ref, wxT, bh_ref[...], tc, bh, d, lambda v: v)
    _push_tiles(wgT)
    _gemm_into(g_s, x_ref, wgT, bg_ref[...], tc, bh, d, jax.nn.sigmoid)

    def step(t, carry):
        ha, hb = carry
        _push_tiles(whT)
        for n in (0, 1):
            pltpu.matmul_acc_lhs(0, ha, mxu_index=n, load_staged_rhs=0)
            pltpu.matmul_acc_lhs(0, hb, mxu_index=n, load_staged_rhs=1)
        xh_t = xh_s[t]
        g_t = g_s[t]
        hs_new = []
        for n in (0, 1):
            z = pltpu.matmul_pop(0, (bh, _KT), jnp.float32, mxu_index=n)
            zs = z + xh_t[:, n * _KT:(n + 1) * _KT]
            cand = 2.0 / (1.0 + jnp.exp(-2.0 * zs)) - 1.0
            gn = g_t[:, n * _KT:(n + 1) * _KT]
            hn = (ha, hb)[n]
            h_new = gn * cand + (1.0 - gn) * hn
            out_ref[t, :, n * _KT:(n + 1) * _KT] = h_new
            hs_new.append(h_new)
        return hs_new[0], hs_new[1]

    ha, hb = jax.lax.fori_loop(
        0, tc, step, (h_s[:, 0:_KT], h_s[:, _KT:2 * _KT]))
    h_s[:, 0:_KT] = ha
    h_s[:, _KT:2 * _KT] = hb

    @pl.when(t_idx == n_t - 1)
    def _fin():
        hlast_ref[:, 0:_KT] = ha
        hlast_ref[:, _KT:2 * _KT] = hb


def kernel(x_seq, h0, W_h, W_x, b_h, W_g, b_g):
    B, T, D = x_seq.shape
    Bh = B // _NC
    nT = T // _TC
    x_tm = jnp.swapaxes(x_seq, 0, 1)

    out_tm, h_last = pl.pallas_call(
        _rnn_kernel,
        grid=(_NC, nT),
        in_specs=[
            pl.BlockSpec((_TC, Bh, D), lambda c, t: (t, c, 0)),
            pl.BlockSpec((Bh, D), lambda c, t: (c, 0)),
            pl.BlockSpec((D, D), lambda c, t: (0, 0)),
            pl.BlockSpec((D, D), lambda c, t: (0, 0)),
            pl.BlockSpec((1, D), lambda c, t: (0, 0)),
            pl.BlockSpec((D, D), lambda c, t: (0, 0)),
            pl.BlockSpec((1, D), lambda c, t: (0, 0)),
        ],
        out_specs=[
            pl.BlockSpec((_TC, Bh, D), lambda c, t: (t, c, 0)),
            pl.BlockSpec((Bh, D), lambda c, t: (c, 0)),
        ],
        out_shape=[
            jax.ShapeDtypeStruct((T, B, D), jnp.float32),
            jax.ShapeDtypeStruct((B, D), jnp.float32),
        ],
        scratch_shapes=[
            pltpu.VMEM((Bh, D), jnp.float32),
            pltpu.VMEM((_TC, Bh, D), jnp.float32),
            pltpu.VMEM((_TC, Bh, D), jnp.float32),
        ],
        compiler_params=pltpu.CompilerParams(
            dimension_semantics=("parallel", "arbitrary"),
            vmem_limit_bytes=100 * 1024 * 1024,
        ),
    )(x_tm, h0, W_h.T, W_x.T, b_h.reshape(1, D), W_g.T, b_g.reshape(1, D))

    return jnp.swapaxes(out_tm, 0, 1), h_last


# speedup vs baseline: 1.2635x; 1.2635x over previous
import jax
import jax.numpy as jnp
from jax.experimental import pallas as pl
from jax.experimental.pallas import tpu as pltpu

_TC = 64
_NC = 2
_KT = 256


def _push_tiles(wT):
    for n in (0, 1):
        for k in (0, 1):
            pltpu.matmul_push_rhs(wT[k * _KT:(k + 1) * _KT, n * _KT:(n + 1) * _KT],
                                  staging_register=k, mxu_index=n)


def _gemm_into(scr, xin_s, buf, wT, bias, tc, bh, d, act, first_push):
    rows_per_m = _KT // bh
    num_m = (tc * bh) // _KT
    for m in range(num_m):
        xm = xin_s[buf, m * rows_per_m:(m + 1) * rows_per_m].reshape(_KT, d)
        addr = 0 if m % 2 == 0 else 128
        if m > 0 or first_push:
            _push_tiles(wT)
        for n in (0, 1):
            pltpu.matmul_acc_lhs(addr, xm[:, 0:_KT], mxu_index=n, load_staged_rhs=0)
            pltpu.matmul_acc_lhs(addr, xm[:, _KT:2 * _KT], mxu_index=n, load_staged_rhs=1)
        for n in (0, 1):
            z = pltpu.matmul_pop(addr, (_KT, _KT), jnp.float32, mxu_index=n)
            z = act(z + bias[:, n * _KT:(n + 1) * _KT])
            scr[m * rows_per_m:(m + 1) * rows_per_m, :,
                n * _KT:(n + 1) * _KT] = z.reshape(rows_per_m, bh, _KT)


def _rnn_kernel(x_hbm, h0_ref, whT_ref, wxT_ref, bh_ref, wgT_ref, bg_ref,
                out_hbm, hlast_ref, h_s, xin_s, ost_s, xh_s, g_s,
                in_sem, out_sem):
    tc = _TC
    bh = h0_ref.shape[0]
    d = h0_ref.shape[1]
    c_idx = pl.program_id(0)
    t_idx = pl.program_id(1)
    n_t = pl.num_programs(1)
    row0 = c_idx * bh

    def in_copy(chunk, buf, tt):
        return pltpu.make_async_copy(
            x_hbm.at[pl.ds(row0, bh), chunk * tc + tt, :],
            xin_s.at[buf, tt], in_sem.at[buf])

    def out_copy(chunk, buf, tt):
        return pltpu.make_async_copy(
            ost_s.at[buf, tt],
            out_hbm.at[pl.ds(row0, bh), chunk * tc + tt, :], out_sem.at[buf])

    buf = jax.lax.rem(t_idx, 2)
    nbuf = jax.lax.rem(t_idx + 1, 2)

    @pl.when(t_idx == 0)
    def _init():
        h_s[...] = h0_ref[...]
        for tt in range(tc):
            in_copy(0, 0, tt).start()

    @pl.when(t_idx >= 2)
    def _drain_out():
        for tt in range(tc):
            out_copy(t_idx - 2, buf, tt).wait()

    for tt in range(tc):
        in_copy(t_idx, buf, tt).wait()

    @pl.when(t_idx + 1 < n_t)
    def _prefetch():
        for tt in range(tc):
            in_copy(t_idx + 1, nbuf, tt).start()

    wxT = wxT_ref[...]
    wgT = wgT_ref[...]
    whT = whT_ref[...]

    _push_tiles(wxT)
    _gemm_into(xh_s, xin_s, buf, wxT, bh_ref[...], tc, bh, d, lambda v: v, False)
    _gemm_into(g_s, xin_s, buf, wgT, bg_ref[...], tc, bh, d, jax.nn.sigmoid, True)

    def step(t, carry):
        ha, hb = carry
        _push_tiles(whT)
        for n in (0, 1):
            pltpu.matmul_acc_lhs(0, ha, mxu_index=n, load_staged_rhs=0)
            pltpu.matmul_acc_lhs(0, hb, mxu_index=n, load_staged_rhs=1)
        xh_t = xh_s[t]
        g_t = g_s[t]
        hs_new = []
        for n in (0, 1):
            z = pltpu.matmul_pop(0, (bh, _KT), jnp.float32, mxu_index=n)
            zs = z + xh_t[:, n * _KT:(n + 1) * _KT]
            cand = 2.0 / (1.0 + jnp.exp(-2.0 * zs)) - 1.0
            gn = g_t[:, n * _KT:(n + 1) * _KT]
            hn = (ha, hb)[n]
            h_new = gn * cand + (1.0 - gn) * hn
            ost_s[buf, t, :, n * _KT:(n + 1) * _KT] = h_new
            hs_new.append(h_new)
        return hs_new[0], hs_new[1]

    ha, hb = jax.lax.fori_loop(
        0, tc, step, (h_s[:, 0:_KT], h_s[:, _KT:2 * _KT]))
    h_s[:, 0:_KT] = ha
    h_s[:, _KT:2 * _KT] = hb

    for tt in range(tc):
        out_copy(t_idx, buf, tt).start()

    @pl.when(t_idx == n_t - 1)
    def _fin():
        hlast_ref[:, 0:_KT] = ha
        hlast_ref[:, _KT:2 * _KT] = hb
        @pl.when(t_idx >= 1)
        def _drain_prev():
            for tt in range(tc):
                out_copy(t_idx - 1, nbuf, tt).wait()
        for tt in range(tc):
            out_copy(t_idx, buf, tt).wait()


def kernel(x_seq, h0, W_h, W_x, b_h, W_g, b_g):
    B, T, D = x_seq.shape
    Bh = B // _NC
    nT = T // _TC

    out_all, h_last = pl.pallas_call(
        _rnn_kernel,
        grid=(_NC, nT),
        in_specs=[
            pl.BlockSpec(memory_space=pl.ANY),
            pl.BlockSpec((Bh, D), lambda c, t: (c, 0)),
            pl.BlockSpec((D, D), lambda c, t: (0, 0)),
            pl.BlockSpec((D, D), lambda c, t: (0, 0)),
            pl.BlockSpec((1, D), lambda c, t: (0, 0)),
            pl.BlockSpec((D, D), lambda c, t: (0, 0)),
            pl.BlockSpec((1, D), lambda c, t: (0, 0)),
        ],
        out_specs=[
            pl.BlockSpec(memory_space=pl.ANY),
            pl.BlockSpec((Bh, D), lambda c, t: (c, 0)),
        ],
        out_shape=[
            jax.ShapeDtypeStruct((B, T, D), jnp.float32),
            jax.ShapeDtypeStruct((B, D), jnp.float32),
        ],
        scratch_shapes=[
            pltpu.VMEM((Bh, D), jnp.float32),
            pltpu.VMEM((2, _TC, Bh, D), jnp.float32),
            pltpu.VMEM((2, _TC, Bh, D), jnp.float32),
            pltpu.VMEM((_TC, Bh, D), jnp.float32),
            pltpu.VMEM((_TC, Bh, D), jnp.float32),
            pltpu.SemaphoreType.DMA((2,)),
            pltpu.SemaphoreType.DMA((2,)),
        ],
        compiler_params=pltpu.CompilerParams(
            dimension_semantics=("parallel", "arbitrary"),
            vmem_limit_bytes=100 * 1024 * 1024,
        ),
    )(x_seq, h0, W_h.T, W_x.T, b_h.reshape(1, D), W_g.T, b_g.reshape(1, D))

    return out_all, h_last


# speedup vs baseline: 1.2777x; 1.0112x over previous
import jax
import jax.numpy as jnp
from jax.experimental import pallas as pl
from jax.experimental.pallas import tpu as pltpu

_TC = 128
_NC = 2
_KT = 256


def _push_tiles(wT):
    for n in (0, 1):
        for k in (0, 1):
            pltpu.matmul_push_rhs(wT[k * _KT:(k + 1) * _KT, n * _KT:(n + 1) * _KT],
                                  staging_register=k, mxu_index=n)


def _gemm_into(scr, xin_s, buf, wT, bias, tc, bh, d, act, first_push):
    rows_per_m = _KT // bh
    num_m = (tc * bh) // _KT
    for m in range(num_m):
        xm = xin_s[buf, m * rows_per_m:(m + 1) * rows_per_m].reshape(_KT, d)
        addr = 0 if m % 2 == 0 else 128
        if m > 0 or first_push:
            _push_tiles(wT)
        for n in (0, 1):
            pltpu.matmul_acc_lhs(addr, xm[:, 0:_KT], mxu_index=n, load_staged_rhs=0)
            pltpu.matmul_acc_lhs(addr, xm[:, _KT:2 * _KT], mxu_index=n, load_staged_rhs=1)
        for n in (0, 1):
            z = pltpu.matmul_pop(addr, (_KT, _KT), jnp.float32, mxu_index=n)
            z = act(z + bias[:, n * _KT:(n + 1) * _KT])
            scr[m * rows_per_m:(m + 1) * rows_per_m, :,
                n * _KT:(n + 1) * _KT] = z.reshape(rows_per_m, bh, _KT)


def _rnn_kernel(x_hbm, h0_ref, whT_ref, wxT_ref, bh_ref, wgT_ref, bg_ref,
                out_hbm, hlast_ref, h_s, xin_s, ost_s, xh_s, g_s,
                in_sem, out_sem):
    tc = _TC
    bh = h0_ref.shape[0]
    d = h0_ref.shape[1]
    c_idx = pl.program_id(0)
    t_idx = pl.program_id(1)
    n_t = pl.num_programs(1)
    row0 = c_idx * bh

    def in_copy(chunk, buf, tt):
        return pltpu.make_async_copy(
            x_hbm.at[pl.ds(row0, bh), chunk * tc + tt, :],
            xin_s.at[buf, tt], in_sem.at[buf])

    def out_copy(chunk, buf, tt):
        return pltpu.make_async_copy(
            ost_s.at[buf, tt],
            out_hbm.at[pl.ds(row0, bh), chunk * tc + tt, :], out_sem.at[buf])

    buf = jax.lax.rem(t_idx, 2)
    nbuf = jax.lax.rem(t_idx + 1, 2)

    @pl.when(t_idx == 0)
    def _init():
        h_s[...] = h0_ref[...]
        for tt in range(tc):
            in_copy(0, 0, tt).start()

    @pl.when(t_idx >= 2)
    def _drain_out():
        for tt in range(tc):
            out_copy(t_idx - 2, buf, tt).wait()

    @pl.when(t_idx + 1 < n_t)
    def _prefetch():
        for tt in range(tc):
            in_copy(t_idx + 1, nbuf, tt).start()

    for tt in range(tc):
        in_copy(t_idx, buf, tt).wait()

    wxT = wxT_ref[...]
    wgT = wgT_ref[...]
    whT = whT_ref[...]

    _push_tiles(wxT)
    _gemm_into(xh_s, xin_s, buf, wxT, bh_ref[...], tc, bh, d, lambda v: v, False)
    _gemm_into(g_s, xin_s, buf, wgT, bg_ref[...], tc, bh, d, jax.nn.sigmoid, True)

    def step(t, carry):
        ha, hb = carry
        _push_tiles(whT)
        for n in (0, 1):
            pltpu.matmul_acc_lhs(0, ha, mxu_index=n, load_staged_rhs=0)
            pltpu.matmul_acc_lhs(0, hb, mxu_index=n, load_staged_rhs=1)
        xh_t = xh_s[t]
        g_t = g_s[t]
        hs_new = []
        for n in (0, 1):
            z = pltpu.matmul_pop(0, (bh, _KT), jnp.float32, mxu_index=n)
            zs = z + xh_t[:, n * _KT:(n + 1) * _KT]
            cand = 2.0 / (1.0 + jnp.exp(-2.0 * zs)) - 1.0
            gn = g_t[:, n * _KT:(n + 1) * _KT]
            hn = (ha, hb)[n]
            h_new = gn * cand + (1.0 - gn) * hn
            ost_s[buf, t, :, n * _KT:(n + 1) * _KT] = h_new
            hs_new.append(h_new)
        return hs_new[0], hs_new[1]

    ha, hb = jax.lax.fori_loop(
        0, tc, step, (h_s[:, 0:_KT], h_s[:, _KT:2 * _KT]))
    h_s[:, 0:_KT] = ha
    h_s[:, _KT:2 * _KT] = hb

    for tt in range(tc):
        out_copy(t_idx, buf, tt).start()

    @pl.when(t_idx == n_t - 1)
    def _fin():
        hlast_ref[:, 0:_KT] = ha
        hlast_ref[:, _KT:2 * _KT] = hb
        @pl.when(t_idx >= 1)
        def _drain_prev():
            for tt in range(tc):
                out_copy(t_idx - 1, nbuf, tt).wait()
        for tt in range(tc):
            out_copy(t_idx, buf, tt).wait()


def kernel(x_seq, h0, W_h, W_x, b_h, W_g, b_g):
    B, T, D = x_seq.shape
    Bh = B // _NC
    nT = T // _TC

    out_all, h_last = pl.pallas_call(
        _rnn_kernel,
        grid=(_NC, nT),
        in_specs=[
            pl.BlockSpec(memory_space=pl.ANY),
            pl.BlockSpec((Bh, D), lambda c, t: (c, 0)),
            pl.BlockSpec((D, D), lambda c, t: (0, 0)),
            pl.BlockSpec((D, D), lambda c, t: (0, 0)),
            pl.BlockSpec((1, D), lambda c, t: (0, 0)),
            pl.BlockSpec((D, D), lambda c, t: (0, 0)),
            pl.BlockSpec((1, D), lambda c, t: (0, 0)),
        ],
        out_specs=[
            pl.BlockSpec(memory_space=pl.ANY),
            pl.BlockSpec((Bh, D), lambda c, t: (c, 0)),
        ],
        out_shape=[
            jax.ShapeDtypeStruct((B, T, D), jnp.float32),
            jax.ShapeDtypeStruct((B, D), jnp.float32),
        ],
        scratch_shapes=[
            pltpu.VMEM((Bh, D), jnp.float32),
            pltpu.VMEM((2, _TC, Bh, D), jnp.float32),
            pltpu.VMEM((2, _TC, Bh, D), jnp.float32),
            pltpu.VMEM((_TC, Bh, D), jnp.float32),
            pltpu.VMEM((_TC, Bh, D), jnp.float32),
            pltpu.SemaphoreType.DMA((2,)),
            pltpu.SemaphoreType.DMA((2,)),
        ],
        compiler_params=pltpu.CompilerParams(
            dimension_semantics=("parallel", "arbitrary"),
            vmem_limit_bytes=100 * 1024 * 1024,
        ),
    )(x_seq, h0, W_h.T, W_x.T, b_h.reshape(1, D), W_g.T, b_g.reshape(1, D))

    return out_all, h_last
